# baseline (speedup 1.0000x reference)
import functools

import jax
import jax.numpy as jnp
from jax import lax
from jax.experimental import pallas as pl
from jax.experimental.pallas import tpu as pltpu
from jax.experimental.pallas import tpu_sc as plsc

_N = 10000
_E = 320000
_H = 128
_CHUNK = 40
_NC = 2
_NS = 16
_NW = _NC * _NS
_ROWS_MAIN = 624
_ROWS_TAIL = _N - _NS * _ROWS_MAIN
_NCHUNKS = _E // _CHUNK


@functools.partial(
    pl.kernel,
    out_type=jax.ShapeDtypeStruct((_NC, _N, _H), jnp.float32),
    mesh=plsc.VectorSubcoreMesh(core_axis_name="c", subcore_axis_name="s"),
    scratch_types=[
        pltpu.VMEM((8, _CHUNK), jnp.int32),
        pltpu.VMEM((8, _CHUNK), jnp.int32),
        pltpu.VMEM((_CHUNK, _H), jnp.float32),
        pltpu.VMEM((_CHUNK, _H), jnp.float32),
        pltpu.VMEM((_CHUNK, _H), jnp.float32),
        pltpu.VMEM((_CHUNK, _H), jnp.float32),
        pltpu.VMEM((_CHUNK, _H), jnp.float32),
        pltpu.VMEM((_CHUNK, _H), jnp.float32),
        pltpu.VMEM((_CHUNK, _H), jnp.float32),
        pltpu.VMEM((_CHUNK, _H), jnp.float32),
        pltpu.VMEM_SHARED((_N, _H), jnp.float32),
        pltpu.SemaphoreType.DMA((8,)),
        pltpu.SemaphoreType.DMA((4,)),
        pltpu.SemaphoreType.DMA((4,)),
        pltpu.SemaphoreType.DMA((2,)),
    ],
)
def _sc_edge_phase(h_hbm, ea_hbm, src_hbm, dst_hbm, out_hbm,
                   src_v, dst_v, rows0, rows1, rows2, rows3,
                   ea0, ea1, ea2, ea3, agg_sp,
                   isem, gsem, easem, ssem):
    c = lax.axis_index("c")
    s = lax.axis_index("s")
    wid = c * _NS + s
    rows = (rows0, rows1, rows2, rows3)
    eab = (ea0, ea1, ea2, ea3)

    @pl.loop(0, _CHUNK)
    def _zero_buf(r):
        for k in range(_H // 16):
            rows0[r, pl.ds(k * 16, 16)] = jnp.zeros((16,), jnp.float32)

    row0 = s * _ROWS_MAIN
    off = 0
    while off < _ROWS_MAIN:
        nrows = min(_CHUNK, _ROWS_MAIN - off)
        pltpu.sync_copy(rows0.at[pl.ds(0, nrows)],
                        agg_sp.at[pl.ds(row0 + off, nrows)])
        off += nrows

    @pl.when(s == _NS - 1)
    def _zero_tail():
        pltpu.sync_copy(rows0.at[pl.ds(0, _ROWS_TAIL)],
                        agg_sp.at[pl.ds(_NS * _ROWS_MAIN, _ROWS_TAIL)])

    plsc.subcore_barrier()

    q, r = divmod(_NCHUNKS, _NW)
    my_count = q + jnp.where(wid < r, 1, 0)
    my_start = wid * q + jnp.minimum(wid, r)

    def idx_issue(j, p4):
        base = pl.multiple_of((my_start + j) * _CHUNK, _CHUNK)
        pltpu.async_copy(src_hbm.at[pl.ds(base, _CHUNK)], src_v.at[p4],
                         isem.at[p4])
        pltpu.async_copy(dst_hbm.at[pl.ds(base, _CHUNK)], dst_v.at[p4],
                         isem.at[p4])

    def idx_wait(p4):
        pltpu.make_async_copy(src_hbm.at[pl.ds(0, _CHUNK)], src_v.at[p4],
                              isem.at[p4]).wait()
        pltpu.make_async_copy(dst_hbm.at[pl.ds(0, _CHUNK)], dst_v.at[p4],
                              isem.at[p4]).wait()

    def fetch_issue(j, p8, p4):
        base = pl.multiple_of((my_start + j) * _CHUNK, _CHUNK)
        pltpu.async_copy(h_hbm.at[src_v.at[p8]], rows[p4], gsem.at[p4])
        pltpu.async_copy(ea_hbm.at[pl.ds(base, _CHUNK)], eab[p4],
                         easem.at[p4])

    def fetch_wait(p4):
        pltpu.make_async_copy(h_hbm.at[src_v.at[0]], rows[p4],
                              gsem.at[p4]).wait()
        pltpu.make_async_copy(ea_hbm.at[pl.ds(0, _CHUNK)], eab[p4],
                              easem.at[p4]).wait()

    def scat_wait(p2):
        pltpu.make_async_copy(rows[p2], agg_sp.at[dst_v.at[0]],
                              ssem.at[p2]).wait()

    for jj in range(4):
        idx_issue(jj, jj)
    for jj in range(3):
        idx_wait(jj)
        fetch_issue(jj, jj, jj)

    @pl.loop(0, (_NCHUNKS // _NW + 7) // 8)
    def _outer(t):
        for b in range(8):
            j = t * 8 + b
            p8 = b
            p4 = b % 4
            p2 = b % 2

            @pl.when(j < my_count)
            def _chunk():
                fetch_wait(p4)

                @pl.when(j >= 1)
                def _buf_free():
                    scat_wait((p2 + 1) % 2)

                @pl.when(j + 3 < my_count)
                def _next_fetch():
                    idx_wait((p8 + 3) % 8)
                    fetch_issue(j + 3, (p8 + 3) % 8, (p4 + 3) % 4)

                @pl.when(j + 4 < my_count)
                def _next_idx():
                    idx_issue(j + 4, (p8 + 4) % 8)

                @pl.loop(0, _CHUNK)
                def _relu_rows(rr):
                    for k in range(_H // 16):
                        sl = pl.ds(k * 16, 16)
                        rows[p4][rr, sl] = jnp.maximum(
                            rows[p4][rr, sl] + eab[p4][rr, sl], 0.0)

                pltpu.async_copy(rows[p4], agg_sp.at[dst_v.at[p8]],
                                 ssem.at[p2], add=True)

    @pl.when(my_count % 2 == 1)
    def _drain_even():
        scat_wait(0)

    @pl.when(my_count % 2 == 0)
    def _drain_odd():
        scat_wait(1)

    plsc.subcore_barrier()
    pltpu.sync_copy(agg_sp.at[pl.ds(row0, _ROWS_MAIN)],
                    out_hbm.at[c, pl.ds(row0, _ROWS_MAIN)])

    @pl.when(s == _NS - 1)
    def _write_tail():
        pltpu.sync_copy(agg_sp.at[pl.ds(_NS * _ROWS_MAIN, _ROWS_TAIL)],
                        out_hbm.at[c, pl.ds(_NS * _ROWS_MAIN, _ROWS_TAIL)])


def _linear_body(x_ref, w_ref, b_ref, o_ref):
    o_ref[...] = (
        jnp.dot(x_ref[...], w_ref[...], preferred_element_type=jnp.float32)
        + b_ref[...]
    )


def _linear(x, w, b, block_rows):
    n, d = x.shape
    h_out = w.shape[1]
    return pl.pallas_call(
        _linear_body,
        grid=(n // block_rows,),
        in_specs=[
            pl.BlockSpec((block_rows, d), lambda i: (i, 0)),
            pl.BlockSpec((d, h_out), lambda i: (0, 0)),
            pl.BlockSpec((1, h_out), lambda i: (0, 0)),
        ],
        out_specs=pl.BlockSpec((block_rows, h_out), lambda i: (i, 0)),
        out_shape=jax.ShapeDtypeStruct((n, h_out), jnp.float32),
    )(x, w, b.reshape(1, -1))


def _mlp_body(h_ref, a0_ref, a1_ref, w1_ref, b1_ref, w2_ref, b2_ref, o_ref):
    z = h_ref[...] + a0_ref[...] + a1_ref[...]
    z = jnp.maximum(
        jnp.dot(z, w1_ref[...], preferred_element_type=jnp.float32)
        + b1_ref[...], 0.0)
    o_ref[...] = jnp.maximum(
        jnp.dot(z, w2_ref[...], preferred_element_type=jnp.float32)
        + b2_ref[...], 0.0)


def _mlp(h, a0, a1, w1, b1, w2, b2, block_rows=1000):
    n, d = h.shape
    row_spec = pl.BlockSpec((block_rows, d), lambda i: (i, 0))
    mat_spec = pl.BlockSpec((d, d), lambda i: (0, 0))
    vec_spec = pl.BlockSpec((1, d), lambda i: (0, 0))
    return pl.pallas_call(
        _mlp_body,
        grid=(n // block_rows,),
        in_specs=[row_spec, row_spec, row_spec,
                  mat_spec, vec_spec, mat_spec, vec_spec],
        out_specs=row_spec,
        out_shape=jax.ShapeDtypeStruct((n, d), jnp.float32),
    )(h, a0, a1, w1, b1.reshape(1, -1), w2, b2.reshape(1, -1))


def kernel(x, edge_index, batch, edge_attr, params):
    src = edge_index[0]
    dst = edge_index[1]
    h = _linear(x, params["W_ne2"], params["b_ne2"], block_rows=1000)
    ea = _linear(edge_attr, params["W_ee"], params["b_ee"], block_rows=3200)
    for lp in params["layers"]:
        scale = lp["g"] / jnp.sqrt(1.0 + 1e-5)
        w1 = lp["W1"] * scale[None, :]
        b1 = lp["b1"] * scale + lp["be"]
        agg = _sc_edge_phase(h, ea, src, dst)
        h = _mlp(h, agg[0], agg[1], w1, b1, lp["W2"], lp["b2"])
    return h


# speedup vs baseline: 1.0436x; 1.0436x over previous
import functools

import jax
import jax.numpy as jnp
from jax import lax
from jax.experimental import pallas as pl
from jax.experimental.pallas import tpu as pltpu
from jax.experimental.pallas import tpu_sc as plsc

_N = 10000
_E = 320000
_H = 128
_CHUNK = 64
_NC = 2
_NS = 16
_NW = _NC * _NS
_ROWS_MAIN = 624
_ROWS_TAIL = _N - _NS * _ROWS_MAIN
_NCHUNKS = _E // _CHUNK


@functools.partial(
    pl.kernel,
    out_type=jax.ShapeDtypeStruct((_NC, _N, _H), jnp.float32),
    mesh=plsc.VectorSubcoreMesh(core_axis_name="c", subcore_axis_name="s"),
    scratch_types=[
        pltpu.VMEM((4, _CHUNK), jnp.int32),
        pltpu.VMEM((4, _CHUNK), jnp.int32),
        pltpu.VMEM((_CHUNK, _H), jnp.float32),
        pltpu.VMEM((_CHUNK, _H), jnp.float32),
        pltpu.VMEM((_CHUNK, _H), jnp.float32),
        pltpu.VMEM((_CHUNK, _H), jnp.float32),
        pltpu.VMEM((_CHUNK, _H), jnp.float32),
        pltpu.VMEM((_CHUNK, _H), jnp.float32),
        pltpu.VMEM_SHARED((_N, _H), jnp.float32),
        pltpu.SemaphoreType.DMA((4,)),
        pltpu.SemaphoreType.DMA((3,)),
        pltpu.SemaphoreType.DMA((3,)),
        pltpu.SemaphoreType.DMA((2,)),
        pltpu.SemaphoreType.DMA,
    ],
)
def _sc_edge_phase(h_hbm, ea_hbm, src_hbm, dst_hbm, out_hbm,
                   src_v, dst_v, rows0, rows1, rows2, ea0, ea1, ea2, agg_sp,
                   isem, gsem, easem, ssem, zsem):
    c = lax.axis_index("c")
    s = lax.axis_index("s")
    wid = c * _NS + s
    rows = (rows0, rows1, rows2)
    eab = (ea0, ea1, ea2)

    q, r = divmod(_NCHUNKS, _NW)
    my_count = q + jnp.where(wid < r, 1, 0)
    my_start = wid * q + jnp.minimum(wid, r)

    def idx_issue(j, p4):
        base = pl.multiple_of((my_start + j) * _CHUNK, _CHUNK)
        pltpu.async_copy(src_hbm.at[pl.ds(base, _CHUNK)], src_v.at[p4],
                         isem.at[p4])
        pltpu.async_copy(dst_hbm.at[pl.ds(base, _CHUNK)], dst_v.at[p4],
                         isem.at[p4])

    def idx_wait(p4):
        pltpu.make_async_copy(src_hbm.at[pl.ds(0, _CHUNK)], src_v.at[p4],
                              isem.at[p4]).wait()
        pltpu.make_async_copy(dst_hbm.at[pl.ds(0, _CHUNK)], dst_v.at[p4],
                              isem.at[p4]).wait()

    def fetch_issue(j, p4, p3):
        base = pl.multiple_of((my_start + j) * _CHUNK, _CHUNK)
        pltpu.async_copy(h_hbm.at[src_v.at[p4]], rows[p3], gsem.at[p3])
        pltpu.async_copy(ea_hbm.at[pl.ds(base, _CHUNK)], eab[p3],
                         easem.at[p3])

    def fetch_wait(p3):
        pltpu.make_async_copy(h_hbm.at[src_v.at[0]], rows[p3],
                              gsem.at[p3]).wait()
        pltpu.make_async_copy(ea_hbm.at[pl.ds(0, _CHUNK)], eab[p3],
                              easem.at[p3]).wait()

    def scat_wait(p2):
        pltpu.make_async_copy(rows[p2 % 3], agg_sp.at[dst_v.at[0]],
                              ssem.at[p2]).wait()

    idx_issue(0, 0)
    idx_issue(1, 1)
    idx_issue(2, 2)

    @pl.loop(0, _CHUNK)
    def _zero_buf(zr):
        for k in range(_H // 16):
            rows2[zr, pl.ds(k * 16, 16)] = jnp.zeros((16,), jnp.float32)

    idx_wait(0)
    fetch_issue(0, 0, 0)
    idx_wait(1)
    fetch_issue(1, 1, 1)

    row0 = s * _ROWS_MAIN
    zcopies = []
    off = 0
    while off < _ROWS_MAIN:
        nrows = min(_CHUNK, _ROWS_MAIN - off)
        zcopies.append((pl.ds(0, nrows), pl.ds(row0 + off, nrows)))
        off += nrows
    for zsrc, zdst in zcopies:
        pltpu.async_copy(rows2.at[zsrc], agg_sp.at[zdst], zsem)

    @pl.when(s == _NS - 1)
    def _zero_tail():
        pltpu.async_copy(rows2.at[pl.ds(0, _ROWS_TAIL)],
                         agg_sp.at[pl.ds(_NS * _ROWS_MAIN, _ROWS_TAIL)],
                         zsem)

    for zsrc, zdst in zcopies:
        pltpu.make_async_copy(rows2.at[zsrc], agg_sp.at[zdst], zsem).wait()

    @pl.when(s == _NS - 1)
    def _zero_tail_wait():
        pltpu.make_async_copy(rows2.at[pl.ds(0, _ROWS_TAIL)],
                              agg_sp.at[pl.ds(_NS * _ROWS_MAIN, _ROWS_TAIL)],
                              zsem).wait()

    plsc.subcore_barrier()

    @pl.loop(0, (_NCHUNKS // _NW + 1 + 11) // 12)
    def _outer(t):
        for b in range(12):
            j = t * 12 + b
            p4 = b % 4
            p3 = b % 3
            p2 = b % 2

            @pl.when(j < my_count)
            def _chunk():
                fetch_wait(p3)

                @pl.when(j >= 1)
                def _buf_free():
                    scat_wait((p2 + 1) % 2)

                @pl.when(j + 2 < my_count)
                def _next_fetch():
                    idx_wait((p4 + 2) % 4)
                    fetch_issue(j + 2, (p4 + 2) % 4, (p3 + 2) % 3)

                @pl.when(j + 3 < my_count)
                def _next_idx():
                    idx_issue(j + 3, (p4 + 3) % 4)

                @pl.loop(0, _CHUNK)
                def _relu_rows(rr):
                    for k in range(_H // 16):
                        sl = pl.ds(k * 16, 16)
                        rows[p3][rr, sl] = jnp.maximum(
                            rows[p3][rr, sl] + eab[p3][rr, sl], 0.0)

                pltpu.async_copy(rows[p3], agg_sp.at[dst_v.at[p4]],
                                 ssem.at[p2], add=True)

    @pl.when(my_count % 2 == 1)
    def _drain_even():
        scat_wait(0)

    @pl.when(my_count % 2 == 0)
    def _drain_odd():
        scat_wait(1)

    plsc.subcore_barrier()
    pltpu.sync_copy(agg_sp.at[pl.ds(row0, _ROWS_MAIN)],
                    out_hbm.at[c, pl.ds(row0, _ROWS_MAIN)])

    @pl.when(s == _NS - 1)
    def _write_tail():
        pltpu.sync_copy(agg_sp.at[pl.ds(_NS * _ROWS_MAIN, _ROWS_TAIL)],
                        out_hbm.at[c, pl.ds(_NS * _ROWS_MAIN, _ROWS_TAIL)])


def _linear_body(x_ref, w_ref, b_ref, o_ref):
    o_ref[...] = (
        jnp.dot(x_ref[...], w_ref[...], preferred_element_type=jnp.float32)
        + b_ref[...]
    )


def _linear(x, w, b, block_rows):
    n, d = x.shape
    h_out = w.shape[1]
    return pl.pallas_call(
        _linear_body,
        grid=(n // block_rows,),
        in_specs=[
            pl.BlockSpec((block_rows, d), lambda i: (i, 0)),
            pl.BlockSpec((d, h_out), lambda i: (0, 0)),
            pl.BlockSpec((1, h_out), lambda i: (0, 0)),
        ],
        out_specs=pl.BlockSpec((block_rows, h_out), lambda i: (i, 0)),
        out_shape=jax.ShapeDtypeStruct((n, h_out), jnp.float32),
    )(x, w, b.reshape(1, -1))


def _mlp_body(h_ref, a0_ref, a1_ref, w1_ref, b1_ref, w2_ref, b2_ref, o_ref):
    z = h_ref[...] + a0_ref[...] + a1_ref[...]
    z = jnp.maximum(
        jnp.dot(z, w1_ref[...], preferred_element_type=jnp.float32)
        + b1_ref[...], 0.0)
    o_ref[...] = jnp.maximum(
        jnp.dot(z, w2_ref[...], preferred_element_type=jnp.float32)
        + b2_ref[...], 0.0)


def _mlp(h, a0, a1, w1, b1, w2, b2, block_rows=1000):
    n, d = h.shape
    row_spec = pl.BlockSpec((block_rows, d), lambda i: (i, 0))
    mat_spec = pl.BlockSpec((d, d), lambda i: (0, 0))
    vec_spec = pl.BlockSpec((1, d), lambda i: (0, 0))
    return pl.pallas_call(
        _mlp_body,
        grid=(n // block_rows,),
        in_specs=[row_spec, row_spec, row_spec,
                  mat_spec, vec_spec, mat_spec, vec_spec],
        out_specs=row_spec,
        out_shape=jax.ShapeDtypeStruct((n, d), jnp.float32),
    )(h, a0, a1, w1, b1.reshape(1, -1), w2, b2.reshape(1, -1))


def kernel(x, edge_index, batch, edge_attr, params):
    src = edge_index[0]
    dst = edge_index[1]
    h = _linear(x, params["W_ne2"], params["b_ne2"], block_rows=1000)
    ea = _linear(edge_attr, params["W_ee"], params["b_ee"], block_rows=3200)
    for lp in params["layers"]:
        scale = lp["g"] / jnp.sqrt(1.0 + 1e-5)
        w1 = lp["W1"] * scale[None, :]
        b1 = lp["b1"] * scale + lp["be"]
        agg = _sc_edge_phase(h, ea, src, dst)
        h = _mlp(h, agg[0], agg[1], w1, b1, lp["W2"], lp["b2"])
    return h
